# Initial kernel scaffold; baseline (speedup 1.0000x reference)
import jax
import jax.numpy as jnp
from jax.experimental import pallas as pl


def kernel(rois, roi_cls_loc, roi_score):
    raise NotImplementedError("write your pallas kernel here")


# speedup vs baseline: 3.9424x; 3.9424x over previous
import jax
import jax.numpy as jnp
from jax.experimental import pallas as pl

N_CLASS = 21
N_ROI = 1000
NP = 1024
IMG_H, IMG_W = 600.0, 800.0
SCORE_LOW = 0.05
NMS_THRESH = 0.3
JT = 128


def _body(rois_n, rois_T, loc_n, loc_T, sc_n, sc_T, boxes_out, scores_out):
    c = pl.program_id(0)

    sn = sc_n[...]
    col = jax.lax.broadcasted_iota(jnp.int32, (NP, 32), 1)
    valid = col < N_CLASS
    m = jnp.max(jnp.where(valid, sn, -1e30), axis=1, keepdims=True)
    e = jnp.where(valid, jnp.exp(sn - m), 0.0)
    denom = jnp.sum(e, axis=1, keepdims=True)
    si = jnp.sum(jnp.where(col == (c + 1), e, 0.0), axis=1, keepdims=True) / denom
    si = jnp.where(si > SCORE_LOW, si, 0.0)

    st = sc_T[...]
    row = jax.lax.broadcasted_iota(jnp.int32, (24, NP), 0)
    validt = row < N_CLASS
    mt = jnp.max(jnp.where(validt, st, -1e30), axis=0, keepdims=True)
    et = jnp.where(validt, jnp.exp(st - mt), 0.0)
    denomt = jnp.sum(et, axis=0, keepdims=True)
    sj = jnp.sum(jnp.where(row == (c + 1), et, 0.0), axis=0, keepdims=True) / denomt
    sj = jnp.where(sj > SCORE_LOW, sj, 0.0)

    rn = rois_n[...]

    def coln(a, k):
        return jax.lax.slice(a, (0, k), (a.shape[0], k + 1))

    ry1, rx1, ry2, rx2 = coln(rn, 0), coln(rn, 1), coln(rn, 2), coln(rn, 3)
    ln = loc_n[0]
    dy, dx, dh, dw = coln(ln, 0), coln(ln, 1), coln(ln, 2), coln(ln, 3)
    sh = ry2 - ry1
    sw = rx2 - rx1
    cy = dy * sh + (ry1 + 0.5 * sh)
    cx = dx * sw + (rx1 + 0.5 * sw)
    hh = jnp.exp(dh) * sh
    ww = jnp.exp(dw) * sw
    y1i = jnp.clip(cy - 0.5 * hh, 0.0, IMG_H)
    x1i = jnp.clip(cx - 0.5 * ww, 0.0, IMG_W)
    y2i = jnp.clip(cy + 0.5 * hh, 0.0, IMG_H)
    x2i = jnp.clip(cx + 0.5 * ww, 0.0, IMG_W)
    area_i = jnp.maximum(y2i - y1i, 0.0) * jnp.maximum(x2i - x1i, 0.0)

    rt = rois_T[...]

    def rowt(a, k):
        return jax.lax.slice(a, (k, 0), (k + 1, a.shape[1]))

    ty1, tx1, ty2, tx2 = rowt(rt, 0), rowt(rt, 1), rowt(rt, 2), rowt(rt, 3)
    lt = loc_T[0]
    tdy, tdx, tdh, tdw = rowt(lt, 0), rowt(lt, 1), rowt(lt, 2), rowt(lt, 3)
    tsh = ty2 - ty1
    tsw = tx2 - tx1
    tcy = tdy * tsh + (ty1 + 0.5 * tsh)
    tcx = tdx * tsw + (tx1 + 0.5 * tsw)
    thh = jnp.exp(tdh) * tsh
    tww = jnp.exp(tdw) * tsw
    y1j = jnp.clip(tcy - 0.5 * thh, 0.0, IMG_H)
    x1j = jnp.clip(tcx - 0.5 * tww, 0.0, IMG_W)
    y2j = jnp.clip(tcy + 0.5 * thh, 0.0, IMG_H)
    x2j = jnp.clip(tcx + 0.5 * tww, 0.0, IMG_W)
    area_j = jnp.maximum(y2j - y1j, 0.0) * jnp.maximum(x2j - x1j, 0.0)

    iidx = jax.lax.broadcasted_iota(jnp.int32, (NP, 1), 0)
    supp = jnp.zeros((NP, 1), dtype=jnp.bool_)
    for t in range(NP // JT):
        def jt(a):
            return jax.lax.slice(a, (0, t * JT), (1, (t + 1) * JT))

        jy1, jx1, jy2, jx2 = jt(y1j), jt(x1j), jt(y2j), jt(x2j)
        ja, js = jt(area_j), jt(sj)
        iy1 = jnp.maximum(y1i, jy1)
        ix1 = jnp.maximum(x1i, jx1)
        iy2 = jnp.minimum(y2i, jy2)
        ix2 = jnp.minimum(x2i, jx2)
        inter = jnp.maximum(iy2 - iy1, 0.0) * jnp.maximum(ix2 - ix1, 0.0)
        union = area_i + ja - inter
        iou = inter / jnp.maximum(union, 1e-8)
        jidx = jax.lax.broadcasted_iota(jnp.int32, (1, JT), 1) + t * JT
        higher = ((js > si) | ((js == si) & (jidx < iidx))) & (jidx != iidx)
        bad = higher & (iou > NMS_THRESH)
        supp = supp | jnp.any(bad, axis=1, keepdims=True)

    keep = jnp.logical_not(supp) & (si > SCORE_LOW)
    outs = jnp.where(keep, si, 0.0)

    boxes_out[...] = jnp.concatenate(
        [y1i, x1i, y2i, x2i, jnp.zeros((NP, 4), jnp.float32)], axis=1
    ).reshape(1, NP, 8)
    scores_out[...] = jnp.concatenate(
        [outs, jnp.zeros((NP, 7), jnp.float32)], axis=1
    ).reshape(1, NP, 8)


def kernel(rois, roi_cls_loc, roi_score):
    rois = rois.astype(jnp.float32)
    loc = roi_cls_loc.astype(jnp.float32).reshape(N_ROI, N_CLASS, 4)[:, 1:, :]
    sc = roi_score.astype(jnp.float32)

    rois_n = jnp.zeros((NP, 8), jnp.float32).at[:N_ROI, :4].set(rois)
    rois_T = jnp.zeros((8, NP), jnp.float32).at[:4, :N_ROI].set(rois.T)
    loc_t = loc.transpose(1, 0, 2)
    loc_n = jnp.zeros((N_CLASS - 1, NP, 8), jnp.float32).at[:, :N_ROI, :4].set(loc_t)
    loc_T = jnp.zeros((N_CLASS - 1, 8, NP), jnp.float32).at[:, :4, :N_ROI].set(
        loc_t.transpose(0, 2, 1))
    sc_n = jnp.zeros((NP, 32), jnp.float32).at[:N_ROI, :N_CLASS].set(sc)
    sc_T = jnp.zeros((24, NP), jnp.float32).at[:N_CLASS, :N_ROI].set(sc.T)

    grid = (N_CLASS - 1,)
    boxes, scores = pl.pallas_call(
        _body,
        grid=grid,
        in_specs=[
            pl.BlockSpec((NP, 8), lambda c: (0, 0)),
            pl.BlockSpec((8, NP), lambda c: (0, 0)),
            pl.BlockSpec((1, NP, 8), lambda c: (c, 0, 0)),
            pl.BlockSpec((1, 8, NP), lambda c: (c, 0, 0)),
            pl.BlockSpec((NP, 32), lambda c: (0, 0)),
            pl.BlockSpec((24, NP), lambda c: (0, 0)),
        ],
        out_specs=[
            pl.BlockSpec((1, NP, 8), lambda c: (c, 0, 0)),
            pl.BlockSpec((1, NP, 8), lambda c: (c, 0, 0)),
        ],
        out_shape=[
            jax.ShapeDtypeStruct((N_CLASS - 1, NP, 8), jnp.float32),
            jax.ShapeDtypeStruct((N_CLASS - 1, NP, 8), jnp.float32),
        ],
    )(rois_n, rois_T, loc_n, loc_T, sc_n, sc_T)

    return boxes[:, :N_ROI, :4], scores[:, :N_ROI, 0]


# speedup vs baseline: 7.2141x; 1.8299x over previous
import functools
import jax
import jax.numpy as jnp
from jax import lax
from jax.experimental import pallas as pl
from jax.experimental.pallas import tpu as pltpu, tpu_sc as plsc

N_CLASS = 21
N_ROI = 1000
NP = 1024
CAP = 1040
IMG_H, IMG_W = 600.0, 800.0
SCORE_LOW = 0.05
NMS_THRESH = 0.3
L = 16

_mesh = plsc.VectorSubcoreMesh(core_axis_name="c", subcore_axis_name="s")


@functools.partial(
    pl.kernel,
    mesh=_mesh,
    compiler_params=pltpu.CompilerParams(needs_layout_passes=False),
    out_type=[
        jax.ShapeDtypeStruct((N_CLASS - 1, 4, NP), jnp.float32),
        jax.ShapeDtypeStruct((N_CLASS - 1, NP), jnp.float32),
    ],
    scratch_types=[
        pltpu.VMEM((4, NP), jnp.float32),
        pltpu.VMEM((4, NP), jnp.float32),
        pltpu.VMEM((N_CLASS, NP), jnp.float32),
        pltpu.VMEM((4, NP), jnp.float32),
        pltpu.VMEM((CAP,), jnp.float32),
        pltpu.VMEM((CAP,), jnp.float32),
        pltpu.VMEM((CAP,), jnp.float32),
        pltpu.VMEM((CAP,), jnp.float32),
        pltpu.VMEM((CAP,), jnp.float32),
        pltpu.VMEM((CAP,), jnp.float32),
        pltpu.VMEM((CAP,), jnp.int32),
        pltpu.VMEM((NP,), jnp.float32),
    ],
)
def _sc_nms(rois_hbm, loc_hbm, sc_hbm, boxes_out, scores_out,
            rois_v, loc_v, sc_v, box_v, y1c, x1c, y2c, x2c, areac, scc,
            idxc, out_s):
    core = lax.axis_index("c")
    sub = lax.axis_index("s")
    cls = core * 10 + sub

    @pl.when(sub < 10)
    def _():
        pltpu.sync_copy(rois_hbm, rois_v)
        pltpu.sync_copy(loc_hbm.at[cls], loc_v)
        pltpu.sync_copy(sc_hbm, sc_v)

        lane = lax.broadcasted_iota(jnp.int32, (L,), 0)

        def chunk_body(k, cnt):
            sl = pl.ds(k * L, L)
            def mx_body(cc, acc):
                return jnp.maximum(acc, sc_v[cc, sl])
            m = lax.fori_loop(0, N_CLASS, mx_body, jnp.full((L,), -jnp.inf, jnp.float32))

            def sum_body(cc, acc):
                return acc + jnp.exp(sc_v[cc, sl] - m)
            denom = lax.fori_loop(0, N_CLASS, sum_body, jnp.zeros((L,), jnp.float32))
            s = jnp.exp(sc_v[cls + 1, sl] - m) / denom
            s = jnp.where(s > SCORE_LOW, s, 0.0)
            roi_id = lane + k * L
            s = jnp.where(roi_id < N_ROI, s, 0.0)

            ry1 = rois_v[0, sl]
            rx1 = rois_v[1, sl]
            ry2 = rois_v[2, sl]
            rx2 = rois_v[3, sl]
            sh = ry2 - ry1
            sw = rx2 - rx1
            cy = loc_v[0, sl] * sh + (ry1 + 0.5 * sh)
            cx = loc_v[1, sl] * sw + (rx1 + 0.5 * sw)
            hh = jnp.exp(loc_v[2, sl]) * sh
            ww = jnp.exp(loc_v[3, sl]) * sw
            y1 = jnp.minimum(jnp.maximum(cy - 0.5 * hh, 0.0), IMG_H)
            x1 = jnp.minimum(jnp.maximum(cx - 0.5 * ww, 0.0), IMG_W)
            y2 = jnp.minimum(jnp.maximum(cy + 0.5 * hh, 0.0), IMG_H)
            x2 = jnp.minimum(jnp.maximum(cx + 0.5 * ww, 0.0), IMG_W)
            area = jnp.maximum(y2 - y1, 0.0) * jnp.maximum(x2 - x1, 0.0)

            box_v[0, sl] = y1
            box_v[1, sl] = x1
            box_v[2, sl] = y2
            box_v[3, sl] = x2
            out_s[sl] = jnp.zeros((L,), jnp.float32)

            msk = s > 0.0
            csl = pl.ds(cnt, L)
            plsc.store_compressed(y1c.at[csl], y1, mask=msk)
            plsc.store_compressed(x1c.at[csl], x1, mask=msk)
            plsc.store_compressed(y2c.at[csl], y2, mask=msk)
            plsc.store_compressed(x2c.at[csl], x2, mask=msk)
            plsc.store_compressed(areac.at[csl], area, mask=msk)
            plsc.store_compressed(scc.at[csl], s, mask=msk)
            plsc.store_compressed(idxc.at[csl], roi_id, mask=msk)
            npop = plsc.all_reduce_population_count(msk)
            return cnt + npop[0]

        cnt = lax.fori_loop(0, NP // L, chunk_body, jnp.int32(0))
        scc[pl.ds(cnt, L)] = jnp.zeros((L,), jnp.float32)

        nch = lax.shift_right_logical(cnt + (L - 1), 4)

        def ichunk_body(t, _):
            isl = pl.ds(t * L, L)
            iy1 = y1c[isl]
            ix1 = x1c[isl]
            iy2 = y2c[isl]
            ix2 = x2c[isl]
            ia = areac[isl]
            si = scc[isl]
            ipos = lane + t * L

            def j_body(j, supp):
                jv = jnp.full((L,), j, jnp.int32)
                jy1 = plsc.load_gather(y1c, [jv])
                jx1 = plsc.load_gather(x1c, [jv])
                jy2 = plsc.load_gather(y2c, [jv])
                jx2 = plsc.load_gather(x2c, [jv])
                ja = plsc.load_gather(areac, [jv])
                sj = plsc.load_gather(scc, [jv])
                yy1 = jnp.maximum(iy1, jy1)
                xx1 = jnp.maximum(ix1, jx1)
                yy2 = jnp.minimum(iy2, jy2)
                xx2 = jnp.minimum(ix2, jx2)
                inter = jnp.maximum(yy2 - yy1, 0.0) * jnp.maximum(xx2 - xx1, 0.0)
                union = ia + ja - inter
                iou = inter / jnp.maximum(union, 1e-8)
                higher = (sj > si) | ((sj == si) & (jv < ipos))
                return supp | (higher & (iou > NMS_THRESH))

            supp = lax.fori_loop(0, cnt, j_body, jnp.zeros((L,), jnp.bool_))
            outv = jnp.where(supp, 0.0, si)
            kmask = ipos < cnt
            plsc.store_scatter(out_s, [idxc[isl]], outv, mask=kmask)
            return 0

        lax.fori_loop(0, nch, ichunk_body, 0)

        pltpu.sync_copy(box_v, boxes_out.at[cls])
        pltpu.sync_copy(out_s, scores_out.at[cls])


def kernel(rois, roi_cls_loc, roi_score):
    rois = rois.astype(jnp.float32)
    loc = roi_cls_loc.astype(jnp.float32).reshape(N_ROI, N_CLASS, 4)[:, 1:, :]
    sc = roi_score.astype(jnp.float32)

    rois_T = jnp.zeros((4, NP), jnp.float32).at[:, :N_ROI].set(rois.T)
    loc_T = jnp.zeros((N_CLASS - 1, 4, NP), jnp.float32).at[:, :, :N_ROI].set(
        loc.transpose(1, 2, 0))
    sc_T = jnp.zeros((N_CLASS, NP), jnp.float32).at[:, :N_ROI].set(sc.T)

    boxes, scores = _sc_nms(rois_T, loc_T, sc_T)
    return boxes.transpose(0, 2, 1)[:, :N_ROI, :], scores[:, :N_ROI]
